# baseline (speedup 1.0000x reference)
import jax
import jax.numpy as jnp
from jax.experimental import pallas as pl
from jax.experimental.pallas import tpu as pltpu

N_CLASS = 21
N_FG = N_CLASS - 1
N_ROI = 1000
N_PAD = 1024
IMG_H, IMG_W = 600, 800
SCORE_LOW = 0.05
NMS_THRESH = 0.3
R_SCALE = NMS_THRESH / (1.0 + NMS_THRESH)
J_CHUNK = 256
NEG = -1e30


def _decode(sy1, sx1, sy2, sx2, dy, dx, dh, dw):
    src_h = sy2 - sy1
    src_w = sx2 - sx1
    src_cy = sy1 + 0.5 * src_h
    src_cx = sx1 + 0.5 * src_w
    cy = dy * src_h + src_cy
    cx = dx * src_w + src_cx
    h = jnp.exp(dh) * src_h
    w = jnp.exp(dw) * src_w
    by1 = jnp.clip(cy - 0.5 * h, 0.0, float(IMG_H))
    bx1 = jnp.clip(cx - 0.5 * w, 0.0, float(IMG_W))
    by2 = jnp.clip(cy + 0.5 * h, 0.0, float(IMG_H))
    bx2 = jnp.clip(cx + 0.5 * w, 0.0, float(IMG_W))
    return by1, bx1, by2, bx2


def _nms_kernel(locrow_ref, auxrow_ref, bbox_ref, score_ref):
    sc = auxrow_ref[4:28, :]
    rm = jnp.max(sc, axis=0, keepdims=True)
    es = jnp.exp(sc - rm)
    rs = jnp.sum(es, axis=0, keepdims=True)

    lr = locrow_ref[0]
    dy_r, dx_r, dh_r, dw_r = (lr[k : k + 1, :] for k in range(4))
    ar = auxrow_ref[...]
    sy1_r, sx1_r, sy2_r, sx2_r = (ar[k : k + 1, :] for k in range(4))

    own = lr[4:5, :]
    prob_r = jnp.exp(own - rm) / rs
    s_r = jnp.where(prob_r > SCORE_LOW, prob_r, 0.0)
    by1_r, bx1_r, by2_r, bx2_r = _decode(
        sy1_r, sx1_r, sy2_r, sx2_r, dy_r, dx_r, dh_r, dw_r
    )
    ra_r = R_SCALE * (
        jnp.maximum(by2_r - by1_r, 0.0) * jnp.maximum(bx2_r - bx1_r, 0.0)
    )
    i_idx = jax.lax.broadcasted_iota(jnp.int32, (1, N_PAD), 1)

    pack = jnp.concatenate(
        [by1_r, bx1_r, by2_r, bx2_r, ra_r, s_r, ra_r, s_r], axis=0
    )
    packT = pack.T

    suppressed = jnp.zeros((1, N_PAD), jnp.bool_)
    for j0 in range(0, N_PAD, J_CHUNK):
        tp = packT[j0 : j0 + J_CHUNK, :]
        by1_c, bx1_c, by2_c, bx2_c, ra_c, s_c = (
            tp[:, k : k + 1] for k in range(6)
        )
        j_idx = jax.lax.broadcasted_iota(jnp.int32, (J_CHUNK, 1), 0) + j0

        iy1 = jnp.maximum(by1_c, by1_r)
        ix1 = jnp.maximum(bx1_c, bx1_r)
        iy2 = jnp.minimum(by2_c, by2_r)
        ix2 = jnp.minimum(bx2_c, bx2_r)
        inter = jnp.maximum(iy2 - iy1, 0.0) * jnp.maximum(ix2 - ix1, 0.0)
        over = inter > (ra_c + ra_r)
        higher = (s_c > s_r) | ((s_c == s_r) & (j_idx < i_idx))
        supp = jnp.any(over & higher, axis=0, keepdims=True)
        suppressed = suppressed | supp

    keep = jnp.logical_not(suppressed) & (s_r > SCORE_LOW)
    out_s = jnp.where(keep, s_r, 0.0)

    bbox_ref[0] = packT[:N_ROI, 0:4]
    score_ref[0] = out_s[:, :N_ROI]


@jax.jit
def kernel(rois, roi_cls_loc, roi_score):
    f = jnp.float32
    loc3 = roi_cls_loc.reshape(N_ROI, N_CLASS, 4)

    locrow = jnp.full((N_FG, 8, N_PAD), NEG, f)
    locrow = locrow.at[:, 0:4, :N_ROI].set(loc3[:, 1:, :].transpose(1, 2, 0))
    locrow = locrow.at[:, 4, :N_ROI].set(roi_score[:, 1:].T)
    auxrow = jnp.full((32, N_PAD), NEG, f)
    auxrow = auxrow.at[0:4, :N_ROI].set(rois.T)
    auxrow = auxrow.at[4 : 4 + N_CLASS, :N_ROI].set(roi_score.T)

    bboxes, scores = pl.pallas_call(
        _nms_kernel,
        grid=(N_FG,),
        in_specs=[
            pl.BlockSpec((1, 8, N_PAD), lambda c: (c, 0, 0)),
            pl.BlockSpec((32, N_PAD), lambda c: (0, 0)),
        ],
        out_specs=[
            pl.BlockSpec((1, N_ROI, 4), lambda c: (c, 0, 0)),
            pl.BlockSpec((1, 1, N_ROI), lambda c: (c, 0, 0)),
        ],
        out_shape=[
            jax.ShapeDtypeStruct((N_FG, N_ROI, 4), f),
            jax.ShapeDtypeStruct((N_FG, 1, N_ROI), f),
        ],
        compiler_params=pltpu.CompilerParams(
            dimension_semantics=("parallel",)
        ),
    )(locrow, auxrow)

    return bboxes, scores[:, 0, :]


# speedup vs baseline: 1.0454x; 1.0454x over previous
import jax
import jax.numpy as jnp
from jax.experimental import pallas as pl
from jax.experimental.pallas import tpu as pltpu

N_CLASS = 21
N_FG = N_CLASS - 1
N_ROI = 1000
N_PAD = 1024
IMG_H, IMG_W = 600, 800
SCORE_LOW = 0.05
NMS_THRESH = 0.3
R_SCALE = NMS_THRESH / (1.0 + NMS_THRESH)
J_CHUNK = 256
NEG = -1e30


def _decode(sy1, sx1, sy2, sx2, dy, dx, dh, dw):
    src_h = sy2 - sy1
    src_w = sx2 - sx1
    src_cy = sy1 + 0.5 * src_h
    src_cx = sx1 + 0.5 * src_w
    cy = dy * src_h + src_cy
    cx = dx * src_w + src_cx
    h = jnp.exp(dh) * src_h
    w = jnp.exp(dw) * src_w
    by1 = jnp.clip(cy - 0.5 * h, 0.0, float(IMG_H))
    bx1 = jnp.clip(cx - 0.5 * w, 0.0, float(IMG_W))
    by2 = jnp.clip(cy + 0.5 * h, 0.0, float(IMG_H))
    bx2 = jnp.clip(cx + 0.5 * w, 0.0, float(IMG_W))
    return by1, bx1, by2, bx2


def _nms_kernel(
    rois_ref, loc_ref, score_ref, bbox_out, score_out, roisT_s, locT_s, prob_s
):
    c = pl.program_id(0)

    @pl.when(c == 0)
    def _():
        roisT_s[...] = rois_ref[...].T
        locT_s[...] = loc_ref[...].T
        st = score_ref[...].T
        sc = st[0:24, :]
        rm = jnp.max(sc, axis=0, keepdims=True)
        es = jnp.exp(sc - rm)
        rs = jnp.sum(es, axis=0, keepdims=True)
        prob_s[...] = es / rs

    ld = locT_s[pl.ds(8 * (c + 1), 8), :]
    dy_r, dx_r, dh_r, dw_r = (ld[k : k + 1, :] for k in range(4))
    rt = roisT_s[...]
    sy1_r, sx1_r, sy2_r, sx2_r = (rt[k : k + 1, :] for k in range(4))

    prob_r = prob_s[pl.ds(1 + c, 1), :]
    s_r = jnp.where(prob_r > SCORE_LOW, prob_r, 0.0)
    by1_r, bx1_r, by2_r, bx2_r = _decode(
        sy1_r, sx1_r, sy2_r, sx2_r, dy_r, dx_r, dh_r, dw_r
    )
    ra_r = R_SCALE * (
        jnp.maximum(by2_r - by1_r, 0.0) * jnp.maximum(bx2_r - bx1_r, 0.0)
    )
    i_idx = jax.lax.broadcasted_iota(jnp.int32, (1, N_PAD), 1)

    pack = jnp.concatenate(
        [by1_r, bx1_r, by2_r, bx2_r, ra_r, s_r, ra_r, s_r], axis=0
    )
    packT = pack.T

    suppressed = jnp.zeros((1, N_PAD), jnp.bool_)
    for j0 in range(0, N_PAD, J_CHUNK):
        tp = packT[j0 : j0 + J_CHUNK, :]
        by1_c, bx1_c, by2_c, bx2_c, ra_c, s_c = (
            tp[:, k : k + 1] for k in range(6)
        )
        j_idx = jax.lax.broadcasted_iota(jnp.int32, (J_CHUNK, 1), 0) + j0

        iy1 = jnp.maximum(by1_c, by1_r)
        ix1 = jnp.maximum(bx1_c, bx1_r)
        iy2 = jnp.minimum(by2_c, by2_r)
        ix2 = jnp.minimum(bx2_c, bx2_r)
        inter = jnp.maximum(iy2 - iy1, 0.0) * jnp.maximum(ix2 - ix1, 0.0)
        over = inter > (ra_c + ra_r)
        higher = (s_c > s_r) | ((s_c == s_r) & (j_idx < i_idx))
        supp = jnp.any(over & higher, axis=0, keepdims=True)
        suppressed = suppressed | supp

    keep = jnp.logical_not(suppressed) & (s_r > SCORE_LOW)
    out_s = jnp.where(keep, s_r, 0.0)

    bbox_out[0] = packT[:N_ROI, 0:4]
    score_out[0] = out_s[:, :N_ROI]


@jax.jit
def kernel(rois, roi_cls_loc, roi_score):
    f = jnp.float32
    rois_p = jnp.pad(rois, ((0, N_PAD - N_ROI), (0, 4)), constant_values=NEG)
    loc_p = jnp.pad(
        roi_cls_loc.reshape(N_ROI, N_CLASS, 4),
        ((0, N_PAD - N_ROI), (0, 1), (0, 4)),
        constant_values=NEG,
    ).reshape(N_PAD, 8 * (N_CLASS + 1))
    score_p = jnp.pad(
        roi_score, ((0, N_PAD - N_ROI), (0, 32 - N_CLASS)),
        constant_values=NEG,
    )

    bboxes, scores = pl.pallas_call(
        _nms_kernel,
        grid=(N_FG,),
        in_specs=[
            pl.BlockSpec((N_PAD, 8), lambda c: (0, 0)),
            pl.BlockSpec((N_PAD, 176), lambda c: (0, 0)),
            pl.BlockSpec((N_PAD, 32), lambda c: (0, 0)),
        ],
        out_specs=[
            pl.BlockSpec((1, N_ROI, 4), lambda c: (c, 0, 0)),
            pl.BlockSpec((1, 1, N_ROI), lambda c: (c, 0, 0)),
        ],
        out_shape=[
            jax.ShapeDtypeStruct((N_FG, N_ROI, 4), f),
            jax.ShapeDtypeStruct((N_FG, 1, N_ROI), f),
        ],
        scratch_shapes=[
            pltpu.VMEM((8, N_PAD), f),
            pltpu.VMEM((176, N_PAD), f),
            pltpu.VMEM((24, N_PAD), f),
        ],
    )(rois_p, loc_p, score_p)

    return bboxes, scores[:, 0, :]


# speedup vs baseline: 1.0956x; 1.0481x over previous
import jax
import jax.numpy as jnp
from jax.experimental import pallas as pl
from jax.experimental.pallas import tpu as pltpu

N_CLASS = 21
N_FG = N_CLASS - 1
N_ROI = 1000
N_PAD = 1024
IMG_H, IMG_W = 600, 800
SCORE_LOW = 0.05
NMS_THRESH = 0.3
R_SCALE = NMS_THRESH / (1.0 + NMS_THRESH)
J_CHUNK = 256
NEG = -1e30


def _region_any(cols, rows, lo, hi, mode, jlt):
    by1_c, bx1_c, by2_c, bx2_c, ra_c, s_c = cols
    by1_r, bx1_r, by2_r, bx2_r, ra_r, s_r = (r[:, lo:hi] for r in rows)
    iy1 = jnp.maximum(by1_c, by1_r)
    ix1 = jnp.maximum(bx1_c, bx1_r)
    iy2 = jnp.minimum(by2_c, by2_r)
    ix2 = jnp.minimum(bx2_c, bx2_r)
    inter = jnp.maximum(iy2 - iy1, 0.0) * jnp.maximum(ix2 - ix1, 0.0)
    over = inter > (ra_c + ra_r)
    if mode == "gt":
        higher = s_c > s_r
    elif mode == "ge":
        higher = s_c >= s_r
    else:
        higher = (s_c > s_r) | ((s_c == s_r) & jlt)
    return jnp.any(over & higher, axis=0, keepdims=True)


def _nms_kernel(
    rois_ref, loc_ref, score_ref, bbox_out, score_out, locT_s, prob_s
):
    locT_s[...] = loc_ref[...].T.reshape(N_CLASS + 1, 8, N_PAD)
    st = score_ref[...].T
    sc = st[0:24, :]
    rm = jnp.max(sc, axis=0, keepdims=True)
    es = jnp.exp(sc - rm)
    rs = jnp.sum(es, axis=0, keepdims=True)
    prob_s[...] = es / rs

    roisT = rois_ref[...].T
    sy1, sx1, sy2, sx2 = (roisT[k : k + 1, :] for k in range(4))
    src_h = sy2 - sy1
    src_w = sx2 - sx1
    src_cy = sy1 + 0.5 * src_h
    src_cx = sx1 + 0.5 * src_w

    jlt = jax.lax.broadcasted_iota(
        jnp.int32, (J_CHUNK, 1), 0
    ) < jax.lax.broadcasted_iota(jnp.int32, (1, J_CHUNK), 1)

    def body(i, _):
        cls = i + 1
        ld = locT_s[cls]
        dy, dx, dh, dw = (ld[k : k + 1, :] for k in range(4))
        prob_r = prob_s[pl.ds(cls, 1), :]
        s_r = jnp.where(prob_r > SCORE_LOW, prob_r, 0.0)

        cy = dy * src_h + src_cy
        cx = dx * src_w + src_cx
        h = jnp.exp(dh) * src_h
        w = jnp.exp(dw) * src_w
        by1_r = jnp.clip(cy - 0.5 * h, 0.0, float(IMG_H))
        bx1_r = jnp.clip(cx - 0.5 * w, 0.0, float(IMG_W))
        by2_r = jnp.clip(cy + 0.5 * h, 0.0, float(IMG_H))
        bx2_r = jnp.clip(cx + 0.5 * w, 0.0, float(IMG_W))
        ra_r = R_SCALE * (
            jnp.maximum(by2_r - by1_r, 0.0) * jnp.maximum(bx2_r - bx1_r, 0.0)
        )
        rows = (by1_r, bx1_r, by2_r, bx2_r, ra_r, s_r)

        pack = jnp.concatenate(
            [by1_r, bx1_r, by2_r, bx2_r, ra_r, s_r, ra_r, s_r], axis=0
        )
        packT = pack.T

        supp = []
        for j0 in range(0, N_PAD, J_CHUNK):
            tp = packT[j0 : j0 + J_CHUNK, :]
            cols = tuple(tp[:, k : k + 1] for k in range(6))
            parts = []
            if j0 > 0:
                parts.append(_region_any(cols, rows, 0, j0, "gt", None))
            parts.append(
                _region_any(cols, rows, j0, j0 + J_CHUNK, "band", jlt)
            )
            if j0 + J_CHUNK < N_PAD:
                parts.append(
                    _region_any(cols, rows, j0 + J_CHUNK, N_PAD, "ge", None)
                )
            supp.append(jnp.concatenate(parts, axis=1))
        suppressed = supp[0] | supp[1] | supp[2] | supp[3]

        keep = jnp.logical_not(suppressed) & (s_r > SCORE_LOW)
        out_s = jnp.where(keep, s_r, 0.0)

        bbox_out[pl.ds(i, 1)] = packT[:N_ROI, 0:4].reshape(1, N_ROI, 4)
        score_out[pl.ds(i, 1)] = out_s[:, :N_ROI].reshape(1, 1, N_ROI)
        return 0

    jax.lax.fori_loop(0, N_FG, body, 0)


@jax.jit
def kernel(rois, roi_cls_loc, roi_score):
    f = jnp.float32
    rois_p = jnp.pad(rois, ((0, N_PAD - N_ROI), (0, 4)), constant_values=NEG)
    loc_p = jnp.pad(
        roi_cls_loc.reshape(N_ROI, N_CLASS, 4),
        ((0, N_PAD - N_ROI), (0, 1), (0, 4)),
        constant_values=NEG,
    ).reshape(N_PAD, 8 * (N_CLASS + 1))
    score_p = jnp.pad(
        roi_score, ((0, N_PAD - N_ROI), (0, 32 - N_CLASS)),
        constant_values=NEG,
    )

    bboxes, scores = pl.pallas_call(
        _nms_kernel,
        grid=(1,),
        in_specs=[
            pl.BlockSpec((N_PAD, 8), lambda c: (0, 0)),
            pl.BlockSpec((N_PAD, 176), lambda c: (0, 0)),
            pl.BlockSpec((N_PAD, 32), lambda c: (0, 0)),
        ],
        out_specs=[
            pl.BlockSpec((N_FG, N_ROI, 4), lambda c: (0, 0, 0)),
            pl.BlockSpec((N_FG, 1, N_ROI), lambda c: (0, 0, 0)),
        ],
        out_shape=[
            jax.ShapeDtypeStruct((N_FG, N_ROI, 4), f),
            jax.ShapeDtypeStruct((N_FG, 1, N_ROI), f),
        ],
        scratch_shapes=[
            pltpu.VMEM((N_CLASS + 1, 8, N_PAD), f),
            pltpu.VMEM((24, N_PAD), f),
        ],
    )(rois_p, loc_p, score_p)

    return bboxes, scores[:, 0, :]


# speedup vs baseline: 1.1095x; 1.0127x over previous
import jax
import jax.numpy as jnp
from jax.experimental import pallas as pl
from jax.experimental.pallas import tpu as pltpu

N_CLASS = 21
N_FG = N_CLASS - 1
N_ROI = 1000
N_PAD = 1024
IMG_H, IMG_W = 600, 800
SCORE_LOW = 0.05
NMS_THRESH = 0.3
R_SCALE = NMS_THRESH / (1.0 + NMS_THRESH)
TILE = 128
NEG = -1e30


def _tile_any(cols, rows, i0, mode, jlt):
    by1_c, bx1_c, by2_c, bx2_c, ra_c, s_c = cols
    by1_r, bx1_r, by2_r, bx2_r, ra_r, s_r = (
        r[:, i0 : i0 + TILE] for r in rows
    )
    iy1 = jnp.maximum(by1_c, by1_r)
    ix1 = jnp.maximum(bx1_c, bx1_r)
    iy2 = jnp.minimum(by2_c, by2_r)
    ix2 = jnp.minimum(bx2_c, bx2_r)
    inter = jnp.maximum(iy2 - iy1, 0.0) * jnp.maximum(ix2 - ix1, 0.0)
    over = inter > (ra_c + ra_r)
    if mode == "gt":
        higher = s_c > s_r
    elif mode == "ge":
        higher = s_c >= s_r
    else:
        higher = (s_c > s_r) | ((s_c == s_r) & jlt)
    return jnp.any(over & higher, axis=0, keepdims=True)


def _nms_kernel(
    rois_ref, loc_ref, score_ref, bbox_out, score_out, locT_s, prob_s
):
    locT_s[...] = loc_ref[...].T.reshape(N_CLASS + 1, 8, N_PAD)
    st = score_ref[...].T
    sc = st[0:24, :]
    rm = jnp.max(sc, axis=0, keepdims=True)
    es = jnp.exp(sc - rm)
    rs = jnp.sum(es, axis=0, keepdims=True)
    prob_s[...] = es / rs

    roisT = rois_ref[...].T
    sy1, sx1, sy2, sx2 = (roisT[k : k + 1, :] for k in range(4))
    src_h = sy2 - sy1
    src_w = sx2 - sx1
    src_cy = sy1 + 0.5 * src_h
    src_cx = sx1 + 0.5 * src_w

    jlt = jax.lax.broadcasted_iota(
        jnp.int32, (TILE, 1), 0
    ) < jax.lax.broadcasted_iota(jnp.int32, (1, TILE), 1)

    def body(i, _):
        cls = i + 1
        ld = locT_s[cls]
        dy, dx, dh, dw = (ld[k : k + 1, :] for k in range(4))
        prob_r = prob_s[pl.ds(cls, 1), :]
        s_r = jnp.where(prob_r > SCORE_LOW, prob_r, 0.0)

        cy = dy * src_h + src_cy
        cx = dx * src_w + src_cx
        h = jnp.exp(dh) * src_h
        w = jnp.exp(dw) * src_w
        by1_r = jnp.clip(cy - 0.5 * h, 0.0, float(IMG_H))
        bx1_r = jnp.clip(cx - 0.5 * w, 0.0, float(IMG_W))
        by2_r = jnp.clip(cy + 0.5 * h, 0.0, float(IMG_H))
        bx2_r = jnp.clip(cx + 0.5 * w, 0.0, float(IMG_W))
        ra_r = R_SCALE * (
            jnp.maximum(by2_r - by1_r, 0.0) * jnp.maximum(bx2_r - bx1_r, 0.0)
        )
        rows = (by1_r, bx1_r, by2_r, bx2_r, ra_r, s_r)

        pack = jnp.concatenate(
            [by1_r, bx1_r, by2_r, bx2_r, ra_r, s_r, ra_r, s_r], axis=0
        )
        packT = pack.T

        supp = [None] * (N_PAD // TILE)
        for j0 in range(0, N_PAD, TILE):
            tp = packT[j0 : j0 + TILE, :]
            cols = tuple(tp[:, k : k + 1] for k in range(6))
            for it, i0 in enumerate(range(0, N_PAD, TILE)):
                if j0 == i0:
                    mode = "band"
                elif j0 < i0:
                    mode = "ge"
                else:
                    mode = "gt"
                t = _tile_any(cols, rows, i0, mode, jlt)
                supp[it] = t if supp[it] is None else (supp[it] | t)
        suppressed = jnp.concatenate(supp, axis=1)

        keep = jnp.logical_not(suppressed) & (s_r > SCORE_LOW)
        out_s = jnp.where(keep, s_r, 0.0)

        bbox_out[pl.ds(i, 1)] = packT[:N_ROI, 0:4].reshape(1, N_ROI, 4)
        score_out[pl.ds(i, 1)] = out_s[:, :N_ROI].reshape(1, 1, N_ROI)
        return 0

    jax.lax.fori_loop(0, N_FG, body, 0)


@jax.jit
def kernel(rois, roi_cls_loc, roi_score):
    f = jnp.float32
    rois_p = jnp.pad(rois, ((0, N_PAD - N_ROI), (0, 4)), constant_values=NEG)
    loc_p = jnp.pad(
        roi_cls_loc.reshape(N_ROI, N_CLASS, 4),
        ((0, N_PAD - N_ROI), (0, 1), (0, 4)),
        constant_values=NEG,
    ).reshape(N_PAD, 8 * (N_CLASS + 1))
    score_p = jnp.pad(
        roi_score, ((0, N_PAD - N_ROI), (0, 32 - N_CLASS)),
        constant_values=NEG,
    )

    bboxes, scores = pl.pallas_call(
        _nms_kernel,
        grid=(1,),
        in_specs=[
            pl.BlockSpec((N_PAD, 8), lambda c: (0, 0)),
            pl.BlockSpec((N_PAD, 176), lambda c: (0, 0)),
            pl.BlockSpec((N_PAD, 32), lambda c: (0, 0)),
        ],
        out_specs=[
            pl.BlockSpec((N_FG, N_ROI, 4), lambda c: (0, 0, 0)),
            pl.BlockSpec((N_FG, 1, N_ROI), lambda c: (0, 0, 0)),
        ],
        out_shape=[
            jax.ShapeDtypeStruct((N_FG, N_ROI, 4), f),
            jax.ShapeDtypeStruct((N_FG, 1, N_ROI), f),
        ],
        scratch_shapes=[
            pltpu.VMEM((N_CLASS + 1, 8, N_PAD), f),
            pltpu.VMEM((24, N_PAD), f),
        ],
    )(rois_p, loc_p, score_p)

    return bboxes, scores[:, 0, :]


# speedup vs baseline: 1.1231x; 1.0123x over previous
import jax
import jax.numpy as jnp
from jax.experimental import pallas as pl
from jax.experimental.pallas import tpu as pltpu

N_CLASS = 21
N_FG = N_CLASS - 1
N_ROI = 1000
N_PAD = 1024
IMG_H, IMG_W = 600, 800
SCORE_LOW = 0.05
NMS_THRESH = 0.3
R_SCALE = NMS_THRESH / (1.0 + NMS_THRESH)
TILE = 128
NEG = -1e30


def _tile_any(cols, rows, i0, mode, jlt):
    by1_c, bx1_c, by2_c, bx2_c, ra_c, s_c = cols
    by1_r, bx1_r, by2_r, bx2_r, ra_r, s_r = (
        r[:, i0 : i0 + TILE] for r in rows
    )
    iy1 = jnp.maximum(by1_c, by1_r)
    ix1 = jnp.maximum(bx1_c, bx1_r)
    iy2 = jnp.minimum(by2_c, by2_r)
    ix2 = jnp.minimum(bx2_c, bx2_r)
    inter = jnp.maximum(iy2 - iy1, 0.0) * jnp.maximum(ix2 - ix1, 0.0)
    over = inter > (ra_c + ra_r)
    if mode == "gt":
        higher = s_c > s_r
    elif mode == "ge":
        higher = s_c >= s_r
    else:
        higher = (s_c > s_r) | ((s_c == s_r) & jlt)
    return jnp.any(over & higher, axis=0, keepdims=True)


def _nms_kernel(
    rois_ref, loc_ref, score_ref, bbox_out, score_out, locT_s, st_s, rt_s
):
    locT_s[...] = jnp.full((N_CLASS, 4, N_PAD), NEG, jnp.float32)
    locT_s[:, :, :N_ROI] = loc_ref[...].T.reshape(N_CLASS, 4, N_ROI)
    st_s[...] = jnp.full((24, N_PAD), NEG, jnp.float32)
    st_s[0:N_CLASS, :N_ROI] = score_ref[...].T
    rt_s[...] = jnp.full((8, N_PAD), NEG, jnp.float32)
    rt_s[0:4, :N_ROI] = rois_ref[...].T

    sc = st_s[...]
    rm = jnp.max(sc, axis=0, keepdims=True)
    es = jnp.exp(sc - rm)
    rs = jnp.sum(es, axis=0, keepdims=True)
    prob = es / rs
    st_s[...] = prob

    roisT = rt_s[...]
    sy1, sx1, sy2, sx2 = (roisT[k : k + 1, :] for k in range(4))
    src_h = sy2 - sy1
    src_w = sx2 - sx1
    src_cy = sy1 + 0.5 * src_h
    src_cx = sx1 + 0.5 * src_w

    jlt = jax.lax.broadcasted_iota(
        jnp.int32, (TILE, 1), 0
    ) < jax.lax.broadcasted_iota(jnp.int32, (1, TILE), 1)

    def body(i, _):
        cls = i + 1
        ld = locT_s[cls]
        dy, dx, dh, dw = (ld[k : k + 1, :] for k in range(4))
        prob_r = st_s[pl.ds(cls, 1), :]
        s_r = jnp.where(prob_r > SCORE_LOW, prob_r, 0.0)

        cy = dy * src_h + src_cy
        cx = dx * src_w + src_cx
        h = jnp.exp(dh) * src_h
        w = jnp.exp(dw) * src_w
        by1_r = jnp.clip(cy - 0.5 * h, 0.0, float(IMG_H))
        bx1_r = jnp.clip(cx - 0.5 * w, 0.0, float(IMG_W))
        by2_r = jnp.clip(cy + 0.5 * h, 0.0, float(IMG_H))
        bx2_r = jnp.clip(cx + 0.5 * w, 0.0, float(IMG_W))
        ra_r = R_SCALE * (
            jnp.maximum(by2_r - by1_r, 0.0) * jnp.maximum(bx2_r - bx1_r, 0.0)
        )
        rows = (by1_r, bx1_r, by2_r, bx2_r, ra_r, s_r)

        pack = jnp.concatenate(
            [by1_r, bx1_r, by2_r, bx2_r, ra_r, s_r, ra_r, s_r], axis=0
        )
        packT = pack.T

        supp = [None] * (N_PAD // TILE)
        for j0 in range(0, N_PAD, TILE):
            tp = packT[j0 : j0 + TILE, :]
            cols = tuple(tp[:, k : k + 1] for k in range(6))
            for it, i0 in enumerate(range(0, N_PAD, TILE)):
                if j0 == i0:
                    mode = "band"
                elif j0 < i0:
                    mode = "ge"
                else:
                    mode = "gt"
                t = _tile_any(cols, rows, i0, mode, jlt)
                supp[it] = t if supp[it] is None else (supp[it] | t)
        suppressed = jnp.concatenate(supp, axis=1)

        keep = jnp.logical_not(suppressed) & (s_r > SCORE_LOW)
        out_s = jnp.where(keep, s_r, 0.0)

        bbox_out[pl.ds(i, 1)] = packT[:N_ROI, 0:4].reshape(1, N_ROI, 4)
        score_out[pl.ds(i, 1)] = out_s[:, :N_ROI].reshape(1, 1, N_ROI)
        return 0

    jax.lax.fori_loop(0, N_FG, body, 0)


@jax.jit
def kernel(rois, roi_cls_loc, roi_score):
    f = jnp.float32
    bboxes, scores = pl.pallas_call(
        _nms_kernel,
        grid=(1,),
        in_specs=[
            pl.BlockSpec((N_ROI, 4), lambda c: (0, 0)),
            pl.BlockSpec((N_ROI, 4 * N_CLASS), lambda c: (0, 0)),
            pl.BlockSpec((N_ROI, N_CLASS), lambda c: (0, 0)),
        ],
        out_specs=[
            pl.BlockSpec((N_FG, N_ROI, 4), lambda c: (0, 0, 0)),
            pl.BlockSpec((N_FG, 1, N_ROI), lambda c: (0, 0, 0)),
        ],
        out_shape=[
            jax.ShapeDtypeStruct((N_FG, N_ROI, 4), f),
            jax.ShapeDtypeStruct((N_FG, 1, N_ROI), f),
        ],
        scratch_shapes=[
            pltpu.VMEM((N_CLASS, 4, N_PAD), f),
            pltpu.VMEM((24, N_PAD), f),
            pltpu.VMEM((8, N_PAD), f),
        ],
    )(rois, roi_cls_loc, roi_score)

    return bboxes, scores[:, 0, :]
